# baseline (speedup 1.0000x reference)
import jax
import jax.numpy as jnp
import numpy as np
from jax.experimental import pallas as pl
from jax.experimental.pallas import tpu as pltpu

_TAU = 0.5
_K = 8
_D = 100000
_C = 12544
_NC = 8
_DP = _NC * _C

_KS = (np.uint32(0), np.uint32(1), np.uint32(0 ^ 1 ^ 0x1BD11BDA))
_ROT_A = (13, 15, 26, 6)
_ROT_B = (17, 29, 16, 24)


def _rotl(x, r):
    return (x << np.uint32(r)) | (x >> np.uint32(32 - r))


def _threefry_xor(counts):
    x1 = counts + _KS[1]
    x0 = x1
    x1 = _rotl(x1, _ROT_A[0]) ^ x0
    for r in _ROT_A[1:]:
        x0 = x0 + x1
        x1 = _rotl(x1, r)
        x1 = x1 ^ x0
    x0 = x0 + _KS[1]
    x1 = x1 + np.uint32(_KS[2] + np.uint32(1))
    for i in range(1, 5):
        rots = _ROT_A if i % 2 == 0 else _ROT_B
        for r in rots:
            x0 = x0 + x1
            x1 = _rotl(x1, r)
            x1 = x1 ^ x0
        x0 = x0 + _KS[(i + 1) % 3]
        x1 = x1 + np.uint32(_KS[(i + 2) % 3] + np.uint32(i + 1))
    return x0 ^ x1


def _body(logits_ref, out_ref, e_ref):
    b = pl.program_id(0)
    shift = np.float32(2.0) * jnp.max(logits_ref[0]) + np.float32(32.0)
    base = (jnp.uint32(b) * jnp.uint32(_K * _D)
            + jax.lax.broadcasted_iota(jnp.uint32, (_K, _C), 0) * jnp.uint32(_D)
            + jax.lax.broadcasted_iota(jnp.uint32, (_K, _C), 1))

    def ph1(i, acc):
        flat = base + (i * _C).astype(jnp.uint32)
        bits = _threefry_xor(flat)
        mantissa = (bits >> np.uint32(9)) | np.uint32(0x3F800000)
        f = jax.lax.bitcast_convert_type(mantissa, jnp.float32) - np.float32(1.0)
        u = f + np.float32(1e-12)
        w = jnp.log(u)
        r = jax.lax.rsqrt(w * w)
        q = r * r
        lc = logits_ref[0, i, 0, :]
        a = jnp.exp(np.float32(2.0) * lc - shift)
        e = a[None, :] * q
        e_ref[i] = e
        return acc + e

    acc = jax.lax.fori_loop(
        0, _NC, ph1, jnp.zeros((_K, _C), jnp.float32), unroll=4)
    s_inv = np.float32(1.0) / jnp.sum(acc, axis=1, keepdims=True)

    def ph2(i, carry):
        out_ref[0, i, 0, :] = jnp.max(e_ref[i] * s_inv, axis=0)
        return carry

    jax.lax.fori_loop(0, _NC, ph2, 0, unroll=False)


def kernel(logits):
    batch = logits.shape[0]
    lp = jnp.pad(jnp.squeeze(logits, 1), ((0, 0), (0, _DP - _D)),
                 constant_values=-jnp.inf)
    lp = lp.reshape(batch, _NC, 1, _C)
    out = pl.pallas_call(
        _body,
        grid=(batch,),
        in_specs=[pl.BlockSpec((1, _NC, 1, _C), lambda b: (b, 0, 0, 0))],
        out_specs=pl.BlockSpec((1, _NC, 1, _C), lambda b: (b, 0, 0, 0)),
        out_shape=jax.ShapeDtypeStruct((batch, _NC, 1, _C), jnp.float32),
        scratch_shapes=[pltpu.VMEM((_NC, _K, _C), jnp.float32)],
        compiler_params=pltpu.CompilerParams(
            dimension_semantics=("parallel",)),
    )(lp)
    return out.reshape(batch, 1, _DP)[:, :, :_D]


# speedup vs baseline: 1.0069x; 1.0069x over previous
import jax
import jax.numpy as jnp
import numpy as np
from jax.experimental import pallas as pl
from jax.experimental.pallas import tpu as pltpu

_TAU = 0.5
_K = 8
_D = 100000
_C = 12544
_NC = 8
_DP = _NC * _C

_KS = (np.uint32(0), np.uint32(1), np.uint32(0 ^ 1 ^ 0x1BD11BDA))
_ROT_A = (13, 15, 26, 6)
_ROT_B = (17, 29, 16, 24)


def _rotl(x, r):
    return (x << np.uint32(r)) | (x >> np.uint32(32 - r))


def _threefry_xor(counts):
    x1 = counts + _KS[1]
    x0 = x1
    x1 = _rotl(x1, _ROT_A[0]) ^ x0
    for r in _ROT_A[1:]:
        x0 = x0 + x1
        x1 = _rotl(x1, r)
        x1 = x1 ^ x0
    x0 = x0 + _KS[1]
    x1 = x1 + np.uint32(_KS[2] + np.uint32(1))
    for i in range(1, 5):
        rots = _ROT_A if i % 2 == 0 else _ROT_B
        for r in rots:
            x0 = x0 + x1
            x1 = _rotl(x1, r)
            x1 = x1 ^ x0
        x0 = x0 + _KS[(i + 1) % 3]
        x1 = x1 + np.uint32(_KS[(i + 2) % 3] + np.uint32(i + 1))
    return x0 ^ x1


def _body(logits_ref, out_ref, e_ref):
    b = pl.program_id(0)
    shift = np.float32(2.0) * jnp.max(logits_ref[0]) + np.float32(32.0)
    base = (jnp.uint32(b) * jnp.uint32(_K * _D)
            + jax.lax.broadcasted_iota(jnp.uint32, (_K, _C), 0) * jnp.uint32(_D)
            + jax.lax.broadcasted_iota(jnp.uint32, (_K, _C), 1))

    def ph1(i, acc):
        flat = base + (i * _C).astype(jnp.uint32)
        bits = _threefry_xor(flat)
        mantissa = (bits >> np.uint32(9)) | np.uint32(0x3F800000)
        f = jax.lax.bitcast_convert_type(mantissa, jnp.float32) - np.float32(1.0)
        u = f + np.float32(1e-12)
        w = jnp.log(u)
        q = np.float32(1.0) / (w * w)
        lc = logits_ref[0, i, 0, :]
        a = jnp.exp(np.float32(2.0) * lc - shift)
        e = a[None, :] * q
        e_ref[i] = e
        return acc + e

    acc = jax.lax.fori_loop(
        0, _NC, ph1, jnp.zeros((_K, _C), jnp.float32), unroll=4)
    s_inv = np.float32(1.0) / jnp.sum(acc, axis=1, keepdims=True)

    def ph2(i, carry):
        out_ref[0, i, 0, :] = jnp.max(e_ref[i] * s_inv, axis=0)
        return carry

    jax.lax.fori_loop(0, _NC, ph2, 0, unroll=False)


def kernel(logits):
    batch = logits.shape[0]
    lp = jnp.pad(jnp.squeeze(logits, 1), ((0, 0), (0, _DP - _D)),
                 constant_values=-jnp.inf)
    lp = lp.reshape(batch, _NC, 1, _C)
    out = pl.pallas_call(
        _body,
        grid=(batch,),
        in_specs=[pl.BlockSpec((1, _NC, 1, _C), lambda b: (b, 0, 0, 0))],
        out_specs=pl.BlockSpec((1, _NC, 1, _C), lambda b: (b, 0, 0, 0)),
        out_shape=jax.ShapeDtypeStruct((batch, _NC, 1, _C), jnp.float32),
        scratch_shapes=[pltpu.VMEM((_NC, _K, _C), jnp.float32)],
        compiler_params=pltpu.CompilerParams(
            dimension_semantics=("parallel",)),
    )(lp)
    return out.reshape(batch, 1, _DP)[:, :, :_D]
